# baseline (speedup 1.0000x reference)
import functools

import jax
import jax.numpy as jnp
from jax import lax
from jax.experimental import pallas as pl
from jax.experimental.pallas import tpu as pltpu
from jax.experimental.pallas import tpu_sc as plsc

EMBED_DIM = 128
MAX_N = 8
NUM_BUCKETS = 500000
HASH_BASE = 260
HASH_MOD = 1 << 23
MASK = HASH_MOD - 1

BSZ = 16
SEQ = 2048
NPOS = BSZ * SEQ
NTAB = 1 + (MAX_N - 2)
NW = 32
C = 32
CH_PER_W = NPOS // (NW * C)
P_PER_W = NPOS // NW
NBUF = 3
NVEC = EMBED_DIM // 16
TLEN = P_PER_W + 16
TLEN2 = TLEN + 16
NG = TLEN // 16


def _mod_buckets(x):
    for k in (16, 8, 4, 2, 1):
        kd = k * NUM_BUCKETS
        x = jnp.where(x >= kd, x - kd, x)
    return x


def _sc_body(tok_hbm, main_hbm, shared_hbm, size_hbm, out_hbm, *s):
    idx_v = s[0]
    bufs = tuple(s[1 + b * NTAB:1 + (b + 1) * NTAB] for b in range(NBUF))
    outb = tuple(s[1 + NBUF * NTAB + b] for b in range(NBUF))
    size_v = s[1 + NBUF * NTAB + NBUF]
    tok_v = s[2 + NBUF * NTAB + NBUF]
    hsc = s[3 + NBUF * NTAB + NBUF]
    gsem = s[4 + NBUF * NTAB + NBUF:4 + NBUF * NTAB + 2 * NBUF]
    osem = s[4 + NBUF * NTAB + 2 * NBUF:4 + NBUF * NTAB + 3 * NBUF]

    sid = lax.axis_index("s")
    wid = lax.axis_index("c") * 16 + sid
    base = wid * P_PER_W
    srow = lax.rem(wid, 2) * P_PER_W

    @pl.when(wid > 0)
    def _tok_mid():
        pltpu.sync_copy(tok_hbm.at[pl.ds(base - 16, TLEN)], tok_v)

    @pl.when(wid == 0)
    def _tok_first():
        pltpu.sync_copy(tok_hbm.at[pl.ds(0, P_PER_W)],
                        tok_v.at[pl.ds(16, P_PER_W)])

    pltpu.sync_copy(size_hbm, size_v)

    const = []
    for v in range(NVEC):
        sl = pl.ds(v * 16, 16)
        cv = size_v[0, sl]
        for t in range(1, MAX_N - 2):
            cv = cv + size_v[t, sl]
        const.append(cv)

    def h1_body(g, c):
        hsc[pl.ds(16 + g * 16, 16)] = tok_v[pl.ds(g * 16, 16)]
        return c
    lax.fori_loop(0, NG, h1_body, 0)

    lanes = lax.broadcasted_iota(jnp.int32, (16,), 0)

    def hash_group(g, n, store_idx, b_slot=None, gg=None):
        off = g * 16
        hp = hsc[pl.ds((n - 2) * TLEN2 + 15 + off, 16)]
        tv = tok_v[pl.ds(off, 16)]
        hc = (hp * HASH_BASE + tv) & MASK
        hsc[pl.ds((n - 1) * TLEN2 + 16 + off, 16)] = hc
        if store_idx:
            @pl.when(n >= 3)
            def _():
                x = _mod_buckets(hc)
                s_vec = srow + (off - 16) + lanes
                idx_v[b_slot, n - 2, pl.ds(gg * 16, 16)] = jnp.where(
                    s_vec < n - 1, 0, x)

    def hash_halo():
        def body(n, c):
            hash_group(jnp.int32(0), n, False)
            return c
        lax.fori_loop(2, MAX_N + 1, body, 0)

    def hash_chunk(chd, b):
        for gg in range(2):
            g = 2 * chd + 1 + gg
            sl16 = pl.ds(g * 16, 16)
            idx_v[b, 0, pl.ds(gg * 16, 16)] = tok_v[sl16]

        def body(n, c):
            for gg in range(2):
                hash_group(2 * chd + 1 + gg, n, True, b, gg)
            return c
        lax.fori_loop(2, MAX_N + 1, body, 0)

    def fire_gathers(b):
        for t in range(NTAB):
            table = main_hbm if t == 0 else shared_hbm
            pltpu.async_copy(table.at[idx_v.at[b, t]], bufs[b][t], gsem[b])

    def drain_gathers(b):
        for t in range(NTAB):
            pltpu.make_async_copy(
                shared_hbm.at[pl.ds(0, C)], bufs[b][t], gsem[b]).wait()

    def drain_out(b):
        pltpu.make_async_copy(
            outb[b], out_hbm.at[pl.ds(0, C)], osem[b]).wait()

    hash_halo()
    for ch in range(NBUF):
        hash_chunk(jnp.int32(ch), ch)
        fire_gathers(ch)

    def outer(gi, c):
        for b in range(NBUF):
            ch = gi * NBUF + b

            @pl.when(ch < CH_PER_W)
            def _():
                drain_gathers(b)

                @pl.when(ch >= NBUF)
                def _():
                    drain_out(b)

                def body(r, cc):
                    for v in range(NVEC):
                        sl = pl.ds(v * 16, 16)
                        acc = bufs[b][0][r, sl]
                        for t in range(1, NTAB):
                            acc = acc + bufs[b][t][r, sl]
                        outb[b][r, sl] = (acc + cc[v]) * (1.0 / 7.0)
                    return cc
                lax.fori_loop(0, C, body, tuple(const))
                pltpu.async_copy(
                    outb[b], out_hbm.at[pl.ds(base + ch * C, C)], osem[b])

                @pl.when(ch + NBUF < CH_PER_W)
                def _():
                    hash_chunk(ch + NBUF, b)
                    fire_gathers(b)
        return c

    nit = (CH_PER_W + NBUF - 1) // NBUF
    lax.fori_loop(0, nit, outer, 0)
    for b in range(NBUF):
        drain_out(b)


@functools.cache
def _sc_embed():
    return pl.kernel(
        _sc_body,
        out_type=jax.ShapeDtypeStruct((NPOS, EMBED_DIM), jnp.float32),
        mesh=plsc.VectorSubcoreMesh(core_axis_name="c", subcore_axis_name="s"),
        scratch_types=(
            [pltpu.VMEM((NBUF, NTAB, C), jnp.int32)]
            + [pltpu.VMEM((C, EMBED_DIM), jnp.float32)
               for _ in range(NBUF * NTAB)]
            + [pltpu.VMEM((C, EMBED_DIM), jnp.float32) for _ in range(NBUF)]
            + [pltpu.VMEM((MAX_N - 2, EMBED_DIM), jnp.float32),
               pltpu.VMEM((TLEN,), jnp.int32),
               pltpu.VMEM((MAX_N * TLEN2,), jnp.int32)]
            + [pltpu.SemaphoreType.DMA for _ in range(2 * NBUF)]
        ),
    )


def kernel(tokens, main_w, shared_w, size_w):
    tokens = tokens.astype(jnp.int32).reshape(NPOS)
    out = _sc_embed()(tokens, main_w, shared_w, size_w)
    return out.reshape(BSZ, SEQ, EMBED_DIM)
